# Initial kernel scaffold; baseline (speedup 1.0000x reference)
import jax
import jax.numpy as jnp
from jax.experimental import pallas as pl


def kernel(x, edge_index, W1, b1, g1, be1, W2, b2, g2, be2, W3, b3):
    raise NotImplementedError("write your pallas kernel here")


# speedup vs baseline: 11.1903x; 11.1903x over previous
import functools

import jax
import jax.numpy as jnp
from jax import lax
from jax.experimental import pallas as pl
from jax.experimental.pallas import tpu as pltpu
from jax.experimental.pallas import tpu_sc as plsc

N = 10000
D = 128
E = 320000
NC = 2
NS = 16
NW = NC * NS
CH = 128
NCHUNK = -(-E // (NW * CH))
EPW = NCHUNK * CH
EPAD = EPW * NW
NP = 10240
RPT = NP // NS
_F32 = jnp.float32

_MESH = plsc.VectorSubcoreMesh(core_axis_name="c", subcore_axis_name="s")


@functools.partial(
    pl.kernel,
    out_type=jax.ShapeDtypeStruct((NC, NP, 16), _F32),
    mesh=_MESH,
    scratch_types=[
        pltpu.VMEM((NCHUNK, CH), jnp.int32),
        pltpu.VMEM((CH, 16), _F32),
        pltpu.VMEM((16, 16), _F32),
        pltpu.VMEM_SHARED((NP, 16), _F32),
    ],
)
def _sc_degree(dst_hbm, out_hbm, dst_v, ones_v, zbuf, hist):
    c = lax.axis_index("c")
    s = lax.axis_index("s")
    wid = s * NC + c
    for r in range(16):
        zbuf[r] = jnp.zeros((16,), _F32)
    for r in range(CH):
        ones_v[r] = jnp.ones((16,), _F32)
    row0 = s * RPT
    def _zero(i, carry):
        pltpu.sync_copy(zbuf, hist.at[pl.ds(row0 + i * 16, 16), :])
        return carry
    lax.fori_loop(0, RPT // 16, _zero, 0)
    plsc.subcore_barrier()
    pltpu.sync_copy(dst_hbm.at[wid], dst_v)
    def _acc(j, carry):
        pltpu.sync_copy(ones_v, hist.at[dst_v.at[j]], add=True)
        return carry
    lax.fori_loop(0, NCHUNK, _acc, 0)
    plsc.subcore_barrier()
    pltpu.sync_copy(hist.at[pl.ds(row0, RPT), :],
                    out_hbm.at[c, pl.ds(row0, RPT), :])


@functools.partial(
    pl.kernel,
    out_type=jax.ShapeDtypeStruct((NC, NP, D), _F32),
    mesh=_MESH,
    scratch_types=[
        pltpu.VMEM((NCHUNK, CH), jnp.int32),
        pltpu.VMEM((NCHUNK, CH), jnp.int32),
        pltpu.VMEM((CH, D), _F32),
        pltpu.VMEM((16, D), _F32),
        pltpu.VMEM_SHARED((NP, D), _F32),
        pltpu.SemaphoreType.DMA,
    ],
)
def _sc_aggregate(hs_hbm, src_hbm, dst_hbm, out_hbm,
                  src_v, dst_v, rows_v, zbuf, acc, sem):
    c = lax.axis_index("c")
    s = lax.axis_index("s")
    wid = s * NC + c
    for r in range(16):
        for k in range(D // 16):
            zbuf[r, pl.ds(k * 16, 16)] = jnp.zeros((16,), _F32)
    row0 = s * RPT
    def _zero(i, carry):
        pltpu.sync_copy(zbuf, acc.at[pl.ds(row0 + i * 16, 16), :])
        return carry
    lax.fori_loop(0, RPT // 16, _zero, 0)
    plsc.subcore_barrier()
    pltpu.sync_copy(src_hbm.at[wid], src_v)
    pltpu.sync_copy(dst_hbm.at[wid], dst_v)
    def _edge(j, carry):
        pltpu.async_copy(hs_hbm.at[src_v.at[j]], rows_v, sem).wait()
        pltpu.sync_copy(rows_v, acc.at[dst_v.at[j]], add=True)
        return carry
    lax.fori_loop(0, NCHUNK, _edge, 0)
    plsc.subcore_barrier()
    pltpu.sync_copy(acc.at[pl.ds(row0, RPT), :],
                    out_hbm.at[c, pl.ds(row0, RPT), :])


def _dinv_from(degp):
    deg = degp[0, :, 0] + degp[1, :, 0] + 1.0
    return lax.rsqrt(deg)


def _tc_prep_body(x_ref, w_ref, degp_ref, h_ref, hs_ref):
    dinv = _dinv_from(degp_ref[...])
    h = jnp.dot(x_ref[...], w_ref[...], preferred_element_type=_F32,
                precision=lax.Precision.HIGHEST)
    h_ref[...] = h
    hs_ref[...] = h * dinv[:, None]


_tc_prep = pl.pallas_call(
    _tc_prep_body,
    out_shape=[jax.ShapeDtypeStruct((NP, D), _F32),
               jax.ShapeDtypeStruct((NP, D), _F32)],
)


def _tc_combine_body(p_ref, h_ref, degp_ref, b_ref, z_ref, st_ref):
    dinv = _dinv_from(degp_ref[...])
    h = h_ref[...]
    z = ((p_ref[0] + p_ref[1]) * dinv[:, None]
         + h * (dinv * dinv)[:, None] + b_ref[...][None, :])
    z_ref[...] = z
    rows = lax.broadcasted_iota(jnp.int32, (NP, 1), 0)
    mask = (rows < N).astype(_F32)
    mu = jnp.sum(z * mask, axis=0, keepdims=True) / N
    dz = (z - mu) * mask
    var = jnp.sum(dz * dz, axis=0, keepdims=True) / N
    st_ref[...] = jnp.concatenate([mu, var], axis=0)


_tc_combine = pl.pallas_call(
    _tc_combine_body,
    out_shape=[jax.ShapeDtypeStruct((NP, D), _F32),
               jax.ShapeDtypeStruct((2, D), _F32)],
)


def _tc_norm_mm_body(z_ref, st_ref, degp_ref, g_ref, be_ref, w_ref,
                     hn_ref, hsn_ref):
    dinv = _dinv_from(degp_ref[...])
    mu = st_ref[0][None, :]
    var = st_ref[1][None, :]
    zn = (z_ref[...] - mu) * lax.rsqrt(var + 1e-5) * g_ref[...][None, :] \
        + be_ref[...][None, :]
    rows = lax.broadcasted_iota(jnp.int32, (NP, 1), 0)
    mask = (rows < N).astype(_F32)
    a = jnp.maximum(zn, 0.0) * mask
    hn = jnp.dot(a, w_ref[...], preferred_element_type=_F32,
                 precision=lax.Precision.HIGHEST)
    hn_ref[...] = hn
    hsn_ref[...] = hn * dinv[:, None]


_tc_norm_mm = pl.pallas_call(
    _tc_norm_mm_body,
    out_shape=[jax.ShapeDtypeStruct((NP, D), _F32),
               jax.ShapeDtypeStruct((NP, D), _F32)],
)


def _tc_mid(p, h, degp, b, g, be, w):
    z, st = _tc_combine(p, h, degp, b)
    return _tc_norm_mm(z, st, degp, g, be, w)


def _tc_final_body(p_ref, h_ref, degp_ref, b_ref, out_ref):
    dinv = _dinv_from(degp_ref[...])
    h = h_ref[...]
    out_ref[...] = ((p_ref[0] + p_ref[1]) * dinv[:, None]
                    + h * (dinv * dinv)[:, None] + b_ref[...][None, :])


_tc_final = pl.pallas_call(
    _tc_final_body,
    out_shape=jax.ShapeDtypeStruct((NP, D), _F32),
)


def kernel(x, edge_index, W1, b1, g1, be1, W2, b2, g2, be2, W3, b3):
    xp = jnp.concatenate([x, jnp.zeros((NP - N, D), _F32)], axis=0)
    pad = jnp.full((EPAD - E,), N, dtype=jnp.int32)
    src3 = jnp.concatenate([edge_index[0], pad]).reshape(NW, NCHUNK, CH)
    dst3 = jnp.concatenate([edge_index[1], pad]).reshape(NW, NCHUNK, CH)

    degp = _sc_degree(dst3)
    h1, hs1 = _tc_prep(xp, W1, degp)
    p1 = _sc_aggregate(hs1, src3, dst3)
    h2, hs2 = _tc_mid(p1, h1, degp, b1, g1, be1, W2)
    p2 = _sc_aggregate(hs2, src3, dst3)
    h3, hs3 = _tc_mid(p2, h2, degp, b2, g2, be2, W3)
    p3 = _sc_aggregate(hs3, src3, dst3)
    outp = _tc_final(p3, h3, degp, b3)
    return outp[:N]
